# baseline (speedup 1.0000x reference)
import functools

import jax
import jax.numpy as jnp
from jax import lax
from jax.experimental import pallas as pl
from jax.experimental.pallas import tpu as pltpu
from jax.experimental.pallas import tpu_sc as plsc

N = 256
NC = 2
NS = 16
L = 16
NW = NC * NS
ROWS = N // NW

_C0 = 2.8853900817779268
_C1 = 0.9617966939259756
_C2 = 0.5770780163555854
_C3 = 0.41219858311113246
_C4 = 0.3205988979753252
_SQRT2 = 1.4142135623730951


def _neg_log2(u):
    bits = lax.bitcast_convert_type(u, jnp.int32)
    e = lax.shift_right_logical(bits, 23) - 127
    mbits = lax.bitwise_or(lax.bitwise_and(bits, 0x7FFFFF), 0x3F800000)
    m = lax.bitcast_convert_type(mbits, jnp.float32)
    big = m >= _SQRT2
    m = jnp.where(big, m * 0.5, m)
    ef = e.astype(jnp.float32) + jnp.where(big, 1.0, 0.0)
    r = (m - 1.0) / (m + 1.0)
    r2 = r * r
    p = _C4
    p = p * r2 + _C3
    p = p * r2 + _C2
    p = p * r2 + _C1
    p = p * r2 + _C0
    return -(ef + r * p)


def _gumbel_cond(p, u0, u1):
    l0 = _neg_log2(u0)
    l1 = _neg_log2(u1)
    return l1 >= l0 * jnp.exp(1.0 - 2.0 * p)


@functools.partial(
    pl.kernel,
    out_type=jax.ShapeDtypeStruct((N, N), jnp.float32),
    mesh=plsc.VectorSubcoreMesh(core_axis_name="c", subcore_axis_name="s"),
    scratch_types=[
        pltpu.VMEM((6, ROWS, N), jnp.float32),
        pltpu.VMEM((3, N), jnp.float32),
        pltpu.VMEM((N,), jnp.float32),
        pltpu.VMEM((ROWS, N), jnp.float32),
        pltpu.SemaphoreType.DMA,
        pltpu.SemaphoreType.DMA,
    ],
)
def _dag_sc(edges_h, roots_h, out_h, e_v, rt_v, r_v, out_v, sem_r, sem_e):
    wid = lax.axis_index("s") * NC + lax.axis_index("c")
    base = wid * ROWS
    rows = pl.ds(base, ROWS)

    root_cp = pltpu.async_copy(roots_h, rt_v, sem_r)
    edge_cps = [
        pltpu.async_copy(edges_h.at[k, rows], e_v.at[k], sem_e)
        for k in range(6)
    ]
    root_cp.wait()

    @plsc.parallel_loop(0, N // L, unroll=4)
    def _(k):
        sl = pl.ds(k * L, L)
        cond = _gumbel_cond(rt_v[0, sl], rt_v[1, sl], rt_v[2, sl])
        r_v[sl] = jnp.where(cond, 1.0, 0.0)

    for cp in edge_cps:
        cp.wait()

    lane = lax.iota(jnp.int32, L)
    blk = lax.shift_left(lax.shift_right_logical(base, 4), 4)
    rblk = r_v[pl.ds(blk, L)]
    off = base - blk
    dnums = lax.GatherDimensionNumbers(
        offset_dims=(), collapsed_slice_dims=(0,), start_index_map=(0,))
    for li in range(ROWS):
        gi = base + li
        gi_vec = jnp.full((L,), gi, dtype=jnp.int32)
        off_vec = jnp.full((L,), off + li, dtype=jnp.int32)
        ri_splat = lax.gather(
            rblk, off_vec[:, None], dnums, slice_sizes=(1,),
            mode=lax.GatherScatterMode.PROMISE_IN_BOUNDS)
        one_m_ri = 1.0 - ri_splat

        @plsc.parallel_loop(0, N // L, unroll=4)
        def _(k):
            c0 = k * L
            sl = pl.ds(c0, L)
            a_cond = _gumbel_cond(
                e_v[0, li, sl], e_v[2, li, sl], e_v[3, li, sl])
            bt_cond = _gumbel_cond(
                e_v[1, li, sl], e_v[4, li, sl], e_v[5, li, sl])
            j = lane + c0
            A = jnp.where(a_cond, 1.0 - r_v[sl], 0.0)
            anc = jnp.where(bt_cond & (j < gi_vec), one_m_ri, 0.0)
            out_v[li, sl] = jnp.where(j != gi_vec, A * (1.0 - anc), 0.0)

    pltpu.sync_copy(out_v, out_h.at[rows])


def kernel(root_probs, edge_probs, u_root, u_edge):
    u0 = u_edge[:, :, 0]
    u1 = u_edge[:, :, 1]
    edges = jnp.stack(
        [edge_probs, edge_probs.T, u0, u1, u0.T, u1.T])
    roots = jnp.stack([root_probs, u_root[:, 0], u_root[:, 1]])
    return _dag_sc(edges, roots)


# speedup vs baseline: 1.0285x; 1.0285x over previous
import functools

import jax
import jax.numpy as jnp
from jax import lax
from jax.experimental import pallas as pl
from jax.experimental.pallas import tpu as pltpu
from jax.experimental.pallas import tpu_sc as plsc

N = 256
NC = 2
NS = 16
L = 16
NW = NC * NS
ROWS = N // NW

_C0 = 2.8853900817779268
_C1 = 0.9617966939259756
_C2 = 0.5770780163555854
_C3 = 0.41219858311113246
_C4 = 0.3205988979753252
_SQRT2 = 1.4142135623730951


def _neg_log2(u):
    bits = lax.bitcast_convert_type(u, jnp.int32)
    e = lax.shift_right_logical(bits, 23) - 127
    mbits = lax.bitwise_or(lax.bitwise_and(bits, 0x7FFFFF), 0x3F800000)
    m = lax.bitcast_convert_type(mbits, jnp.float32)
    big = m >= _SQRT2
    m = jnp.where(big, m * 0.5, m)
    ef = e.astype(jnp.float32) + jnp.where(big, 1.0, 0.0)
    r = (m - 1.0) / (m + 1.0)
    r2 = r * r
    p = _C4
    p = p * r2 + _C3
    p = p * r2 + _C2
    p = p * r2 + _C1
    p = p * r2 + _C0
    return -(ef + r * p)


def _gumbel_cond(p, u0, u1):
    l0 = _neg_log2(u0)
    l1 = _neg_log2(u1)
    return l1 >= l0 * jnp.exp(1.0 - 2.0 * p)


@functools.partial(
    pl.kernel,
    out_type=jax.ShapeDtypeStruct((N, N), jnp.float32),
    mesh=plsc.VectorSubcoreMesh(core_axis_name="c", subcore_axis_name="s"),
    scratch_types=[
        pltpu.VMEM((6, ROWS, N), jnp.float32),
        pltpu.VMEM((3 * N,), jnp.float32),
        pltpu.VMEM((N,), jnp.float32),
        pltpu.VMEM((ROWS, N), jnp.float32),
        pltpu.SemaphoreType.DMA,
        pltpu.SemaphoreType.DMA,
    ],
)
def _dag_sc(edges_h, roots_h, out_h, e_v, rt_v, r_v, out_v, sem_r, sem_e):
    wid = lax.axis_index("s") * NC + lax.axis_index("c")
    base = wid * ROWS
    rows = pl.ds(base, ROWS)

    root_cp = pltpu.async_copy(roots_h, rt_v, sem_r)
    edge_cps = [
        pltpu.async_copy(edges_h.at[k, rows], e_v.at[k], sem_e)
        for k in range(6)
    ]
    root_cp.wait()

    @plsc.parallel_loop(0, N // L, unroll=4)
    def _(k):
        c0 = k * L
        cond = _gumbel_cond(rt_v[pl.ds(c0, L)],
                            rt_v[pl.ds(N + c0, L)],
                            rt_v[pl.ds(2 * N + c0, L)])
        r_v[pl.ds(c0, L)] = jnp.where(cond, 1.0, 0.0)

    for cp in edge_cps:
        cp.wait()

    lane = lax.iota(jnp.int32, L)
    blk = lax.shift_left(lax.shift_right_logical(base, 4), 4)
    rblk = r_v[pl.ds(blk, L)]
    off = base - blk
    dnums = lax.GatherDimensionNumbers(
        offset_dims=(), collapsed_slice_dims=(0,), start_index_map=(0,))
    for li in range(ROWS):
        gi = base + li
        gi_vec = jnp.full((L,), gi, dtype=jnp.int32)
        off_vec = jnp.full((L,), off + li, dtype=jnp.int32)
        ri_splat = lax.gather(
            rblk, off_vec[:, None], dnums, slice_sizes=(1,),
            mode=lax.GatherScatterMode.PROMISE_IN_BOUNDS)
        one_m_ri = 1.0 - ri_splat

        @plsc.parallel_loop(0, N // L, unroll=4)
        def _(k):
            c0 = k * L
            sl = pl.ds(c0, L)
            a_cond = _gumbel_cond(
                e_v[0, li, sl], e_v[2, li, sl], e_v[3, li, sl])
            bt_cond = _gumbel_cond(
                e_v[1, li, sl], e_v[4, li, sl], e_v[5, li, sl])
            j = lane + c0
            A = jnp.where(a_cond, 1.0 - r_v[sl], 0.0)
            anc = jnp.where(bt_cond & (j < gi_vec), one_m_ri, 0.0)
            out_v[li, sl] = jnp.where(j != gi_vec, A * (1.0 - anc), 0.0)

    pltpu.sync_copy(out_v, out_h.at[rows])


def kernel(root_probs, edge_probs, u_root, u_edge):
    u01 = u_edge.transpose(2, 0, 1)
    u01T = u_edge.transpose(2, 1, 0)
    edges = jnp.concatenate(
        [edge_probs[None], edge_probs.T[None], u01, u01T])
    roots = jnp.concatenate(
        [root_probs[None], u_root.T]).reshape(3 * N)
    return _dag_sc(edges, roots)


# speedup vs baseline: 1.1051x; 1.0744x over previous
import functools

import jax
import jax.numpy as jnp
from jax import lax
from jax.experimental import pallas as pl
from jax.experimental.pallas import tpu as pltpu
from jax.experimental.pallas import tpu_sc as plsc

N = 256
NC = 2
NS = 16
L = 16
NW = NC * NS
ROWS = N // NW

_C0 = 2.8853900817779268
_C1 = 0.9617966939259756
_C2 = 0.5770780163555854
_C3 = 0.41219858311113246
_OFF = 0x3F330000


def _neg_log2(u):
    ix = lax.bitcast_convert_type(u, jnp.int32) - _OFF
    k = lax.shift_right_arithmetic(ix, 23)
    m = lax.bitcast_convert_type(
        lax.bitwise_and(ix, 0x007FFFFF) + _OFF, jnp.float32)
    r = (m - 1.0) / (m + 1.0)
    r2 = r * r
    p = _C3
    p = p * r2 + _C2
    p = p * r2 + _C1
    p = p * r2 + _C0
    return -(k.astype(jnp.float32) + r * p)


def _gumbel_cond(p, u0, u1):
    l0 = _neg_log2(u0)
    l1 = _neg_log2(u1)
    return l1 >= l0 * jnp.exp(1.0 - 2.0 * p)


@functools.partial(
    pl.kernel,
    out_type=jax.ShapeDtypeStruct((N, N), jnp.float32),
    mesh=plsc.VectorSubcoreMesh(core_axis_name="c", subcore_axis_name="s"),
    scratch_types=[
        pltpu.VMEM((6, ROWS, N), jnp.float32),
        pltpu.VMEM((3 * N,), jnp.float32),
        pltpu.VMEM((N,), jnp.float32),
        pltpu.VMEM((ROWS, N), jnp.float32),
        pltpu.SemaphoreType.DMA,
        pltpu.SemaphoreType.DMA,
    ],
)
def _dag_sc(edges_h, roots_h, out_h, e_v, rt_v, r_v, out_v, sem_r, sem_e):
    wid = lax.axis_index("s") * NC + lax.axis_index("c")
    base = wid * ROWS
    rows = pl.ds(base, ROWS)

    root_cp = pltpu.async_copy(roots_h, rt_v, sem_r)
    edge_cps = [
        pltpu.async_copy(edges_h.at[k, rows], e_v.at[k], sem_e)
        for k in range(6)
    ]
    root_cp.wait()

    @plsc.parallel_loop(0, N // L, unroll=4)
    def _(k):
        c0 = k * L
        cond = _gumbel_cond(rt_v[pl.ds(c0, L)],
                            rt_v[pl.ds(N + c0, L)],
                            rt_v[pl.ds(2 * N + c0, L)])
        r_v[pl.ds(c0, L)] = jnp.where(cond, 0.0, 1.0)

    for cp in edge_cps:
        cp.wait()

    lane = lax.iota(jnp.int32, L)
    blk = lax.shift_left(lax.shift_right_logical(base, 4), 4)
    rblk = r_v[pl.ds(blk, L)]
    off = base - blk
    dnums = lax.GatherDimensionNumbers(
        offset_dims=(), collapsed_slice_dims=(0,), start_index_map=(0,))
    for li in range(ROWS):
        gi = base + li
        gi_vec = jnp.full((L,), gi, dtype=jnp.int32)
        off_vec = jnp.full((L,), off + li, dtype=jnp.int32)
        one_m_ri = lax.gather(
            rblk, off_vec[:, None], dnums, slice_sizes=(1,),
            mode=lax.GatherScatterMode.PROMISE_IN_BOUNDS)

        @plsc.parallel_loop(0, N // L, unroll=4)
        def _(k):
            c0 = k * L
            sl = pl.ds(c0, L)
            a_cond = _gumbel_cond(
                e_v[0, li, sl], e_v[2, li, sl], e_v[3, li, sl])
            bt_cond = _gumbel_cond(
                e_v[1, li, sl], e_v[4, li, sl], e_v[5, li, sl])
            j = lane + c0
            A = jnp.where(a_cond, r_v[sl], 0.0)
            anc = jnp.where(bt_cond & (j < gi_vec), one_m_ri, 0.0)
            out_v[li, sl] = jnp.where(j != gi_vec, A * (1.0 - anc), 0.0)

    pltpu.sync_copy(out_v, out_h.at[rows])


def kernel(root_probs, edge_probs, u_root, u_edge):
    u01 = u_edge.transpose(2, 0, 1)
    u01T = u_edge.transpose(2, 1, 0)
    edges = jnp.concatenate(
        [edge_probs[None], edge_probs.T[None], u01, u01T])
    roots = jnp.concatenate(
        [root_probs[None], u_root.T]).reshape(3 * N)
    return _dag_sc(edges, roots)


# speedup vs baseline: 1.1123x; 1.0065x over previous
import functools

import jax
import jax.numpy as jnp
from jax import lax
from jax.experimental import pallas as pl
from jax.experimental.pallas import tpu as pltpu
from jax.experimental.pallas import tpu_sc as plsc

N = 256
NC = 2
NS = 16
L = 16
NW = NC * NS
ROWS = N // NW

_C0 = 2.8853900817779268
_C1 = 0.9617966939259756
_C2 = 0.5770780163555854
_C3 = 0.41219858311113246
_OFF = 0x3F330000


def _log2(u):
    ix = lax.bitcast_convert_type(u, jnp.int32) - _OFF
    k = lax.shift_right_arithmetic(ix, 23)
    m = lax.bitcast_convert_type(
        lax.bitwise_and(ix, 0x007FFFFF) + _OFF, jnp.float32)
    r = (m - 1.0) / (m + 1.0)
    r2 = r * r
    p = _C3
    p = p * r2 + _C2
    p = p * r2 + _C1
    p = p * r2 + _C0
    return k.astype(jnp.float32) + r * p


def _gumbel_cond(p, u0, u1):
    return _log2(u1) <= _log2(u0) * jnp.exp(1.0 - 2.0 * p)


@functools.partial(
    pl.kernel,
    out_type=jax.ShapeDtypeStruct((N, N), jnp.float32),
    mesh=plsc.VectorSubcoreMesh(core_axis_name="c", subcore_axis_name="s"),
    scratch_types=[
        pltpu.VMEM((6, ROWS, N), jnp.float32),
        pltpu.VMEM((N,), jnp.float32),
        pltpu.VMEM((2, N), jnp.float32),
        pltpu.VMEM((N,), jnp.float32),
        pltpu.VMEM((ROWS, N), jnp.float32),
        pltpu.SemaphoreType.DMA,
        pltpu.SemaphoreType.DMA,
        pltpu.SemaphoreType.DMA,
    ],
)
def _dag_sc(ep_h, epT_h, u01_h, u01T_h, rp_h, urT_h, out_h,
            e_v, rp_v, ur_v, r_v, out_v, sem_r, sem_e, sem_o):
    wid = lax.axis_index("s") * NC + lax.axis_index("c")
    base = wid * ROWS
    rows = pl.ds(base, ROWS)

    rp_cp = pltpu.async_copy(rp_h, rp_v, sem_r)
    ur_cp = pltpu.async_copy(urT_h, ur_v, sem_r)
    edge_cps = [
        pltpu.async_copy(ep_h.at[rows], e_v.at[0], sem_e),
        pltpu.async_copy(epT_h.at[rows], e_v.at[1], sem_e),
        pltpu.async_copy(u01_h.at[0, rows], e_v.at[2], sem_e),
        pltpu.async_copy(u01_h.at[1, rows], e_v.at[3], sem_e),
        pltpu.async_copy(u01T_h.at[0, rows], e_v.at[4], sem_e),
        pltpu.async_copy(u01T_h.at[1, rows], e_v.at[5], sem_e),
    ]
    rp_cp.wait()
    ur_cp.wait()

    @plsc.parallel_loop(0, N // L, unroll=4)
    def _(k):
        sl = pl.ds(k * L, L)
        cond = _gumbel_cond(rp_v[sl], ur_v[0, sl], ur_v[1, sl])
        r_v[sl] = jnp.where(cond, 0.0, 1.0)

    for cp in edge_cps:
        cp.wait()

    lane = lax.iota(jnp.int32, L)
    blk = lax.shift_left(lax.shift_right_logical(base, 4), 4)
    rblk = r_v[pl.ds(blk, L)]
    off = base - blk
    dnums = lax.GatherDimensionNumbers(
        offset_dims=(), collapsed_slice_dims=(0,), start_index_map=(0,))
    out_cps = []
    for li in range(ROWS):
        gi = base + li
        gi_vec = jnp.full((L,), gi, dtype=jnp.int32)
        off_vec = jnp.full((L,), off + li, dtype=jnp.int32)
        one_m_ri = lax.gather(
            rblk, off_vec[:, None], dnums, slice_sizes=(1,),
            mode=lax.GatherScatterMode.PROMISE_IN_BOUNDS)

        @plsc.parallel_loop(0, N // L, unroll=8)
        def _(k):
            c0 = k * L
            sl = pl.ds(c0, L)
            a_cond = _gumbel_cond(
                e_v[0, li, sl], e_v[2, li, sl], e_v[3, li, sl])
            bt_cond = _gumbel_cond(
                e_v[1, li, sl], e_v[4, li, sl], e_v[5, li, sl])
            j = lane + c0
            A = jnp.where(a_cond, r_v[sl], 0.0)
            anc = jnp.where(bt_cond & (j < gi_vec), one_m_ri, 0.0)
            out_v[li, sl] = jnp.where(j != gi_vec, A * (1.0 - anc), 0.0)

        out_cps.append(
            pltpu.async_copy(out_v.at[li], out_h.at[base + li], sem_o))

    for cp in out_cps:
        cp.wait()


def kernel(root_probs, edge_probs, u_root, u_edge):
    u01 = u_edge.transpose(2, 0, 1)
    u01T = u_edge.transpose(2, 1, 0)
    return _dag_sc(edge_probs, edge_probs.T, u01, u01T,
                   root_probs, u_root.T)
